# baseline (speedup 1.0000x reference)
import functools

import jax
import jax.numpy as jnp
from jax.experimental import pallas as pl
from jax.experimental.pallas import tpu as pltpu

EPS_BN = 1e-5
MB = 512


def _stage1_body(x_ref, xold_ref, wup_ref, wlat_ref,
                 zup_ref, zlat_ref, stats_ref):
    b = pl.program_id(0)
    m = pl.program_id(1)

    @pl.when(jnp.logical_and(b == 0, m == 0))
    def _init():
        stats_ref[...] = jnp.zeros_like(stats_ref)

    @pl.when(m == 0)
    def _up():
        zup = jax.lax.dot_general(
            x_ref[0], wup_ref[...], (((1,), (1,)), ((), ())),
            preferred_element_type=jnp.float32)
        zup_ref[0] = zup
        stats_ref[0:1, :] += jnp.sum(zup, axis=0, keepdims=True)
        stats_ref[1:2, :] += jnp.sum(zup * zup, axis=0, keepdims=True)

    zlat = jax.lax.dot_general(
        xold_ref[0], wlat_ref[...], (((1,), (1,)), ((), ())),
        preferred_element_type=jnp.float32)
    zlat_ref[0] = zlat
    stats_ref[2:3, :] += jnp.sum(zlat, axis=0, keepdims=True)
    stats_ref[3:4, :] += jnp.sum(zlat * zlat, axis=0, keepdims=True)


def _stage2_body(zup_ref, zlat_ref, pt_ref, pold_ref, aff_ref,
                 y_ref, h_ref):
    m = pl.program_id(1)

    @pl.when(m == 0)
    def _norm_up():
        h_ref[...] = jnp.maximum(
            zup_ref[0] * aff_ref[0:1, :] + aff_ref[1:2, :], 0.0)

    pold = pold_ref[0]
    pt = pt_ref[0]
    pn = jnp.sum(pt * pt, axis=0, keepdims=True)
    pon = jnp.sum(pold * pold, axis=1, keepdims=True)
    q = jax.lax.dot_general(
        pold * jnp.float32(-2.0), pt, (((1,), (0,)), ((), ())),
        preferred_element_type=jnp.float32,
        precision=jax.lax.Precision.HIGHEST)
    d = q + pn

    inf = jnp.float32(jnp.inf)
    m0 = jnp.min(d, axis=1, keepdims=True)
    s0 = d == m0
    dm = jnp.where(s0, inf, d)
    m1 = jnp.min(dm, axis=1, keepdims=True)
    s1 = dm == m1
    dm = jnp.where(s1, inf, dm)
    m2 = jnp.min(dm, axis=1, keepdims=True)
    s2 = dm == m2

    w0 = 1.0 / jnp.maximum(m0 + pon, 1e-16)
    w1 = 1.0 / jnp.maximum(m1 + pon, 1e-16)
    w2 = 1.0 / jnp.maximum(m2 + pon, 1e-16)
    ws = w0 + w1 + w2
    zero = jnp.zeros_like(d)
    a = jnp.where(s0, w0 / ws, jnp.where(s1, w1 / ws, jnp.where(s2, w2 / ws, zero)))

    interp = jax.lax.dot_general(
        a, h_ref[...], (((1,), (0,)), ((), ())),
        preferred_element_type=jnp.float32)
    lat = jnp.maximum(zlat_ref[0] * aff_ref[2:3, :] + aff_ref[3:4, :], 0.0)
    y_ref[0] = interp + lat


@functools.partial(jax.jit, static_argnames=())
def kernel(x, p, x_old, p_old, W_up, gamma_up, beta_up,
           W_lat, gamma_lat, beta_lat):
    B, N, Cin = x.shape
    M = p_old.shape[1]
    Cout = W_up.shape[0]
    nmb = M // MB

    grid = (B, nmb)
    zup, zlat, stats = pl.pallas_call(
        _stage1_body,
        grid=grid,
        in_specs=[
            pl.BlockSpec((1, N, Cin), lambda b, m: (b, 0, 0)),
            pl.BlockSpec((1, MB, Cout), lambda b, m: (b, m, 0)),
            pl.BlockSpec((Cout, Cin), lambda b, m: (0, 0)),
            pl.BlockSpec((Cout, Cout), lambda b, m: (0, 0)),
        ],
        out_specs=[
            pl.BlockSpec((1, N, Cout), lambda b, m: (b, 0, 0)),
            pl.BlockSpec((1, MB, Cout), lambda b, m: (b, m, 0)),
            pl.BlockSpec((8, Cout), lambda b, m: (0, 0)),
        ],
        out_shape=[
            jax.ShapeDtypeStruct((B, N, Cout), jnp.float32),
            jax.ShapeDtypeStruct((B, M, Cout), jnp.float32),
            jax.ShapeDtypeStruct((8, Cout), jnp.float32),
        ],
        compiler_params=pltpu.CompilerParams(
            dimension_semantics=("arbitrary", "arbitrary")),
    )(x, x_old, W_up, W_lat)

    n_up = jnp.float32(B * N)
    n_lat = jnp.float32(B * M)
    mean_up = stats[0] / n_up
    var_up = jnp.maximum(stats[1] / n_up - mean_up * mean_up, 0.0)
    scale_up = gamma_up * jax.lax.rsqrt(var_up + EPS_BN)
    shift_up = beta_up - mean_up * scale_up
    mean_lat = stats[2] / n_lat
    var_lat = jnp.maximum(stats[3] / n_lat - mean_lat * mean_lat, 0.0)
    scale_lat = gamma_lat * jax.lax.rsqrt(var_lat + EPS_BN)
    shift_lat = beta_lat - mean_lat * scale_lat
    aff = jnp.stack([scale_up, shift_up, scale_lat, shift_lat], axis=0)
    aff = jnp.concatenate([aff, jnp.zeros((4, Cout), jnp.float32)], axis=0)

    p_t = jnp.transpose(p, (0, 2, 1))

    y = pl.pallas_call(
        _stage2_body,
        grid=grid,
        in_specs=[
            pl.BlockSpec((1, N, Cout), lambda b, m: (b, 0, 0)),
            pl.BlockSpec((1, MB, Cout), lambda b, m: (b, m, 0)),
            pl.BlockSpec((1, 3, N), lambda b, m: (b, 0, 0)),
            pl.BlockSpec((1, MB, 3), lambda b, m: (b, m, 0)),
            pl.BlockSpec((8, Cout), lambda b, m: (0, 0)),
        ],
        out_specs=pl.BlockSpec((1, MB, Cout), lambda b, m: (b, m, 0)),
        out_shape=jax.ShapeDtypeStruct((B, M, Cout), jnp.float32),
        scratch_shapes=[pltpu.VMEM((N, Cout), jnp.float32)],
        compiler_params=pltpu.CompilerParams(
            dimension_semantics=("arbitrary", "arbitrary")),
    )(zup, zlat, p_t, p_old, aff)

    return (y, p_old)


# speedup vs baseline: 1.5055x; 1.5055x over previous
import functools

import jax
import jax.numpy as jnp
from jax.experimental import pallas as pl
from jax.experimental.pallas import tpu as pltpu

EPS_BN = 1e-5
MB = 512


def _stage1_body(x_ref, xold_ref, wup_ref, wlat_ref,
                 zup_ref, zlat_ref, stats_ref):
    b = pl.program_id(0)
    m = pl.program_id(1)

    @pl.when(jnp.logical_and(b == 0, m == 0))
    def _init():
        stats_ref[...] = jnp.zeros_like(stats_ref)

    @pl.when(m == 0)
    def _up():
        zup = jax.lax.dot_general(
            x_ref[0], wup_ref[...], (((1,), (1,)), ((), ())),
            preferred_element_type=jnp.float32)
        zup_ref[0] = zup
        stats_ref[0:1, :] += jnp.sum(zup, axis=0, keepdims=True)
        stats_ref[1:2, :] += jnp.sum(zup * zup, axis=0, keepdims=True)

    zlat = jax.lax.dot_general(
        xold_ref[0], wlat_ref[...], (((1,), (1,)), ((), ())),
        preferred_element_type=jnp.float32)
    zlat_ref[0] = zlat
    stats_ref[2:3, :] += jnp.sum(zlat, axis=0, keepdims=True)
    stats_ref[3:4, :] += jnp.sum(zlat * zlat, axis=0, keepdims=True)


def _stage2_body(zup_ref, zlat_ref, pt_ref, pold_ref, aff_ref,
                 y_ref, h_ref):
    m = pl.program_id(1)

    @pl.when(m == 0)
    def _norm_up():
        h_ref[...] = jnp.maximum(
            zup_ref[0] * aff_ref[0:1, :] + aff_ref[1:2, :], 0.0)

    pold = pold_ref[0]
    pt = pt_ref[0]
    d0 = pold[:, 0:1] - pt[0:1, :]
    d1 = pold[:, 1:2] - pt[1:2, :]
    d2c = pold[:, 2:3] - pt[2:3, :]
    d = d0 * d0 + d1 * d1 + d2c * d2c

    inf = jnp.float32(jnp.inf)
    m0 = jnp.min(d, axis=1, keepdims=True)
    s0 = d == m0
    dm = jnp.where(s0, inf, d)
    m1 = jnp.min(dm, axis=1, keepdims=True)
    s1 = dm == m1
    dm = jnp.where(s1, inf, dm)
    m2 = jnp.min(dm, axis=1, keepdims=True)
    s2 = dm == m2

    w0 = 1.0 / jnp.maximum(m0, 1e-16)
    w1 = 1.0 / jnp.maximum(m1, 1e-16)
    w2 = 1.0 / jnp.maximum(m2, 1e-16)
    ws = w0 + w1 + w2
    zero = jnp.zeros_like(d)
    a = jnp.where(s0, w0 / ws, jnp.where(s1, w1 / ws, jnp.where(s2, w2 / ws, zero)))

    interp = jax.lax.dot_general(
        a, h_ref[...], (((1,), (0,)), ((), ())),
        preferred_element_type=jnp.float32)
    lat = jnp.maximum(zlat_ref[0] * aff_ref[2:3, :] + aff_ref[3:4, :], 0.0)
    y_ref[0] = interp + lat


@functools.partial(jax.jit, static_argnames=())
def kernel(x, p, x_old, p_old, W_up, gamma_up, beta_up,
           W_lat, gamma_lat, beta_lat):
    B, N, Cin = x.shape
    M = p_old.shape[1]
    Cout = W_up.shape[0]
    nmb = M // MB

    grid = (B, nmb)
    zup, zlat, stats = pl.pallas_call(
        _stage1_body,
        grid=grid,
        in_specs=[
            pl.BlockSpec((1, N, Cin), lambda b, m: (b, 0, 0)),
            pl.BlockSpec((1, MB, Cout), lambda b, m: (b, m, 0)),
            pl.BlockSpec((Cout, Cin), lambda b, m: (0, 0)),
            pl.BlockSpec((Cout, Cout), lambda b, m: (0, 0)),
        ],
        out_specs=[
            pl.BlockSpec((1, N, Cout), lambda b, m: (b, 0, 0)),
            pl.BlockSpec((1, MB, Cout), lambda b, m: (b, m, 0)),
            pl.BlockSpec((8, Cout), lambda b, m: (0, 0)),
        ],
        out_shape=[
            jax.ShapeDtypeStruct((B, N, Cout), jnp.float32),
            jax.ShapeDtypeStruct((B, M, Cout), jnp.float32),
            jax.ShapeDtypeStruct((8, Cout), jnp.float32),
        ],
        compiler_params=pltpu.CompilerParams(
            dimension_semantics=("arbitrary", "arbitrary")),
    )(x, x_old, W_up, W_lat)

    n_up = jnp.float32(B * N)
    n_lat = jnp.float32(B * M)
    mean_up = stats[0] / n_up
    var_up = jnp.maximum(stats[1] / n_up - mean_up * mean_up, 0.0)
    scale_up = gamma_up * jax.lax.rsqrt(var_up + EPS_BN)
    shift_up = beta_up - mean_up * scale_up
    mean_lat = stats[2] / n_lat
    var_lat = jnp.maximum(stats[3] / n_lat - mean_lat * mean_lat, 0.0)
    scale_lat = gamma_lat * jax.lax.rsqrt(var_lat + EPS_BN)
    shift_lat = beta_lat - mean_lat * scale_lat
    aff = jnp.stack([scale_up, shift_up, scale_lat, shift_lat], axis=0)
    aff = jnp.concatenate([aff, jnp.zeros((4, Cout), jnp.float32)], axis=0)

    p_t = jnp.transpose(p, (0, 2, 1))

    y = pl.pallas_call(
        _stage2_body,
        grid=grid,
        in_specs=[
            pl.BlockSpec((1, N, Cout), lambda b, m: (b, 0, 0)),
            pl.BlockSpec((1, MB, Cout), lambda b, m: (b, m, 0)),
            pl.BlockSpec((1, 3, N), lambda b, m: (b, 0, 0)),
            pl.BlockSpec((1, MB, 3), lambda b, m: (b, m, 0)),
            pl.BlockSpec((8, Cout), lambda b, m: (0, 0)),
        ],
        out_specs=pl.BlockSpec((1, MB, Cout), lambda b, m: (b, m, 0)),
        out_shape=jax.ShapeDtypeStruct((B, M, Cout), jnp.float32),
        scratch_shapes=[pltpu.VMEM((N, Cout), jnp.float32)],
        compiler_params=pltpu.CompilerParams(
            dimension_semantics=("arbitrary", "arbitrary")),
    )(zup, zlat, p_t, p_old, aff)

    return (y, p_old)


# speedup vs baseline: 1.6599x; 1.1026x over previous
import functools

import jax
import jax.numpy as jnp
from jax.experimental import pallas as pl
from jax.experimental.pallas import tpu as pltpu

EPS_BN = 1e-5
MB = 512
MB1 = 2048


def _stage1_body(xf_ref, xoldf_ref, wup_ref, wlat_ref,
                 zupf_ref, zlatf_ref, stats_ref):
    i = pl.program_id(0)

    @pl.when(i == 0)
    def _up():
        zup = jax.lax.dot_general(
            xf_ref[...], wup_ref[...], (((1,), (1,)), ((), ())),
            preferred_element_type=jnp.float32)
        zupf_ref[...] = zup
        stats_ref[...] = jnp.zeros_like(stats_ref)
        stats_ref[0:1, :] = jnp.sum(zup, axis=0, keepdims=True)
        stats_ref[1:2, :] = jnp.sum(zup * zup, axis=0, keepdims=True)

    zlat = jax.lax.dot_general(
        xoldf_ref[...], wlat_ref[...], (((1,), (1,)), ((), ())),
        preferred_element_type=jnp.float32)
    zlatf_ref[...] = zlat
    stats_ref[2:3, :] += jnp.sum(zlat, axis=0, keepdims=True)
    stats_ref[3:4, :] += jnp.sum(zlat * zlat, axis=0, keepdims=True)


def _stage2_body(n_up, n_lat, zup_ref, zlat_ref, pt_ref, pold_ref, stats_ref,
                 gb_ref, y_ref, h_ref, aff_ref):
    b = pl.program_id(0)
    m = pl.program_id(1)

    @pl.when(jnp.logical_and(b == 0, m == 0))
    def _affine():
        mean_up = stats_ref[0:1, :] / n_up
        var_up = jnp.maximum(stats_ref[1:2, :] / n_up - mean_up * mean_up, 0.0)
        scale_up = gb_ref[0:1, :] * jax.lax.rsqrt(var_up + EPS_BN)
        aff_ref[0:1, :] = scale_up
        aff_ref[1:2, :] = gb_ref[1:2, :] - mean_up * scale_up
        mean_lat = stats_ref[2:3, :] / n_lat
        var_lat = jnp.maximum(
            stats_ref[3:4, :] / n_lat - mean_lat * mean_lat, 0.0)
        scale_lat = gb_ref[2:3, :] * jax.lax.rsqrt(var_lat + EPS_BN)
        aff_ref[2:3, :] = scale_lat
        aff_ref[3:4, :] = gb_ref[3:4, :] - mean_lat * scale_lat

    @pl.when(m == 0)
    def _norm_up():
        h_ref[...] = jnp.maximum(
            zup_ref[0] * aff_ref[0:1, :] + aff_ref[1:2, :], 0.0)

    pold = pold_ref[0]
    pt = pt_ref[0]
    d0 = pold[:, 0:1] - pt[0:1, :]
    d1 = pold[:, 1:2] - pt[1:2, :]
    d2c = pold[:, 2:3] - pt[2:3, :]
    d = d0 * d0 + d1 * d1 + d2c * d2c

    inf = jnp.float32(jnp.inf)
    m0 = jnp.min(d, axis=1, keepdims=True)
    s0 = d == m0
    dm = jnp.where(s0, inf, d)
    m1 = jnp.min(dm, axis=1, keepdims=True)
    s1 = dm == m1
    dm = jnp.where(s1, inf, dm)
    m2 = jnp.min(dm, axis=1, keepdims=True)
    s2 = dm == m2

    w0 = 1.0 / jnp.maximum(m0, 1e-16)
    w1 = 1.0 / jnp.maximum(m1, 1e-16)
    w2 = 1.0 / jnp.maximum(m2, 1e-16)
    ws = w0 + w1 + w2
    zero = jnp.zeros_like(d)
    a = jnp.where(s0, w0 / ws, jnp.where(s1, w1 / ws, jnp.where(s2, w2 / ws, zero)))

    interp = jax.lax.dot_general(
        a, h_ref[...], (((1,), (0,)), ((), ())),
        preferred_element_type=jnp.float32)
    lat = jnp.maximum(zlat_ref[0] * aff_ref[2:3, :] + aff_ref[3:4, :], 0.0)
    y_ref[0] = interp + lat


@functools.partial(jax.jit, static_argnames=())
def kernel(x, p, x_old, p_old, W_up, gamma_up, beta_up,
           W_lat, gamma_lat, beta_lat):
    B, N, Cin = x.shape
    M = p_old.shape[1]
    Cout = W_up.shape[0]
    nmb = M // MB

    xf = x.reshape(B * N, Cin)
    xoldf = x_old.reshape(B * M, Cout)
    n1 = (B * M) // MB1

    zupf, zlatf, stats = pl.pallas_call(
        _stage1_body,
        grid=(n1,),
        in_specs=[
            pl.BlockSpec((B * N, Cin), lambda i: (0, 0)),
            pl.BlockSpec((MB1, Cout), lambda i: (i, 0)),
            pl.BlockSpec((Cout, Cin), lambda i: (0, 0)),
            pl.BlockSpec((Cout, Cout), lambda i: (0, 0)),
        ],
        out_specs=[
            pl.BlockSpec((B * N, Cout), lambda i: (0, 0)),
            pl.BlockSpec((MB1, Cout), lambda i: (i, 0)),
            pl.BlockSpec((8, Cout), lambda i: (0, 0)),
        ],
        out_shape=[
            jax.ShapeDtypeStruct((B * N, Cout), jnp.float32),
            jax.ShapeDtypeStruct((B * M, Cout), jnp.float32),
            jax.ShapeDtypeStruct((8, Cout), jnp.float32),
        ],
        compiler_params=pltpu.CompilerParams(
            dimension_semantics=("arbitrary",)),
    )(xf, xoldf, W_up, W_lat)

    zup = zupf.reshape(B, N, Cout)
    zlat = zlatf.reshape(B, M, Cout)
    gb = jnp.stack([gamma_up, beta_up, gamma_lat, beta_lat], axis=0)
    p_t = jnp.transpose(p, (0, 2, 1))

    y = pl.pallas_call(
        functools.partial(_stage2_body, float(B * N), float(B * M)),
        grid=(B, nmb),
        in_specs=[
            pl.BlockSpec((1, N, Cout), lambda b, m: (b, 0, 0)),
            pl.BlockSpec((1, MB, Cout), lambda b, m: (b, m, 0)),
            pl.BlockSpec((1, 3, N), lambda b, m: (b, 0, 0)),
            pl.BlockSpec((1, MB, 3), lambda b, m: (b, m, 0)),
            pl.BlockSpec((8, Cout), lambda b, m: (0, 0)),
            pl.BlockSpec((4, Cout), lambda b, m: (0, 0)),
        ],
        out_specs=pl.BlockSpec((1, MB, Cout), lambda b, m: (b, m, 0)),
        out_shape=jax.ShapeDtypeStruct((B, M, Cout), jnp.float32),
        scratch_shapes=[
            pltpu.VMEM((N, Cout), jnp.float32),
            pltpu.VMEM((8, Cout), jnp.float32),
        ],
        compiler_params=pltpu.CompilerParams(
            dimension_semantics=("arbitrary", "arbitrary")),
    )(zup, zlat, p_t, p_old, stats, gb)

    return (y, p_old)


# speedup vs baseline: 1.7897x; 1.0782x over previous
import functools

import jax
import jax.numpy as jnp
from jax.experimental import pallas as pl
from jax.experimental.pallas import tpu as pltpu

EPS_BN = 1e-5
MB = 1024
MB1 = 2048


def _stage1_body(xf_ref, xoldf_ref, wup_ref, wlat_ref,
                 zupf_ref, zlatf_ref, stats_ref):
    i = pl.program_id(0)

    @pl.when(i == 0)
    def _up():
        zup = jax.lax.dot_general(
            xf_ref[...], wup_ref[...], (((1,), (1,)), ((), ())),
            preferred_element_type=jnp.float32)
        zupf_ref[...] = zup
        stats_ref[...] = jnp.zeros_like(stats_ref)
        stats_ref[0:1, :] = jnp.sum(zup, axis=0, keepdims=True)
        stats_ref[1:2, :] = jnp.sum(zup * zup, axis=0, keepdims=True)

    zlat = jax.lax.dot_general(
        xoldf_ref[...], wlat_ref[...], (((1,), (1,)), ((), ())),
        preferred_element_type=jnp.float32)
    zlatf_ref[...] = zlat
    stats_ref[2:3, :] += jnp.sum(zlat, axis=0, keepdims=True)
    stats_ref[3:4, :] += jnp.sum(zlat * zlat, axis=0, keepdims=True)


def _stage2_body(n_up, n_lat, zup_ref, zlat_ref, pt_ref, pold_ref, stats_ref,
                 gb_ref, y_ref, h_ref, aff_ref):
    b = pl.program_id(0)
    m = pl.program_id(1)

    @pl.when(jnp.logical_and(b == 0, m == 0))
    def _affine():
        mean_up = stats_ref[0:1, :] / n_up
        var_up = jnp.maximum(stats_ref[1:2, :] / n_up - mean_up * mean_up, 0.0)
        scale_up = gb_ref[0:1, :] * jax.lax.rsqrt(var_up + EPS_BN)
        aff_ref[0:1, :] = scale_up
        aff_ref[1:2, :] = gb_ref[1:2, :] - mean_up * scale_up
        mean_lat = stats_ref[2:3, :] / n_lat
        var_lat = jnp.maximum(
            stats_ref[3:4, :] / n_lat - mean_lat * mean_lat, 0.0)
        scale_lat = gb_ref[2:3, :] * jax.lax.rsqrt(var_lat + EPS_BN)
        aff_ref[2:3, :] = scale_lat
        aff_ref[3:4, :] = gb_ref[3:4, :] - mean_lat * scale_lat

    @pl.when(m == 0)
    def _norm_up():
        h_ref[...] = jnp.maximum(
            zup_ref[0] * aff_ref[0:1, :] + aff_ref[1:2, :], 0.0)

    pold = pold_ref[0]
    pt = pt_ref[0]
    d0 = pold[:, 0:1] - pt[0:1, :]
    d1 = pold[:, 1:2] - pt[1:2, :]
    d2c = pold[:, 2:3] - pt[2:3, :]
    d = d0 * d0 + d1 * d1 + d2c * d2c

    inf = jnp.float32(jnp.inf)
    m0 = jnp.min(d, axis=1, keepdims=True)
    s0 = d == m0
    dm = jnp.where(s0, inf, d)
    m1 = jnp.min(dm, axis=1, keepdims=True)
    s1 = dm == m1
    dm = jnp.where(s1, inf, dm)
    m2 = jnp.min(dm, axis=1, keepdims=True)
    s2 = dm == m2

    w0 = 1.0 / jnp.maximum(m0, 1e-16)
    w1 = 1.0 / jnp.maximum(m1, 1e-16)
    w2 = 1.0 / jnp.maximum(m2, 1e-16)
    ws = w0 + w1 + w2
    zero = jnp.zeros_like(d)
    a = jnp.where(s0, w0 / ws, jnp.where(s1, w1 / ws, jnp.where(s2, w2 / ws, zero)))

    interp = jax.lax.dot_general(
        a, h_ref[...], (((1,), (0,)), ((), ())),
        preferred_element_type=jnp.float32)
    lat = jnp.maximum(zlat_ref[0] * aff_ref[2:3, :] + aff_ref[3:4, :], 0.0)
    y_ref[0] = interp + lat


@functools.partial(jax.jit, static_argnames=())
def kernel(x, p, x_old, p_old, W_up, gamma_up, beta_up,
           W_lat, gamma_lat, beta_lat):
    B, N, Cin = x.shape
    M = p_old.shape[1]
    Cout = W_up.shape[0]
    nmb = M // MB

    xf = x.reshape(B * N, Cin)
    xoldf = x_old.reshape(B * M, Cout)
    n1 = (B * M) // MB1

    zupf, zlatf, stats = pl.pallas_call(
        _stage1_body,
        grid=(n1,),
        in_specs=[
            pl.BlockSpec((B * N, Cin), lambda i: (0, 0)),
            pl.BlockSpec((MB1, Cout), lambda i: (i, 0)),
            pl.BlockSpec((Cout, Cin), lambda i: (0, 0)),
            pl.BlockSpec((Cout, Cout), lambda i: (0, 0)),
        ],
        out_specs=[
            pl.BlockSpec((B * N, Cout), lambda i: (0, 0)),
            pl.BlockSpec((MB1, Cout), lambda i: (i, 0)),
            pl.BlockSpec((8, Cout), lambda i: (0, 0)),
        ],
        out_shape=[
            jax.ShapeDtypeStruct((B * N, Cout), jnp.float32),
            jax.ShapeDtypeStruct((B * M, Cout), jnp.float32),
            jax.ShapeDtypeStruct((8, Cout), jnp.float32),
        ],
        compiler_params=pltpu.CompilerParams(
            dimension_semantics=("arbitrary",)),
    )(xf, xoldf, W_up, W_lat)

    zup = zupf.reshape(B, N, Cout)
    zlat = zlatf.reshape(B, M, Cout)
    gb = jnp.stack([gamma_up, beta_up, gamma_lat, beta_lat], axis=0)
    p_t = jnp.transpose(p, (0, 2, 1))

    y = pl.pallas_call(
        functools.partial(_stage2_body, float(B * N), float(B * M)),
        grid=(B, nmb),
        in_specs=[
            pl.BlockSpec((1, N, Cout), lambda b, m: (b, 0, 0)),
            pl.BlockSpec((1, MB, Cout), lambda b, m: (b, m, 0)),
            pl.BlockSpec((1, 3, N), lambda b, m: (b, 0, 0)),
            pl.BlockSpec((1, MB, 3), lambda b, m: (b, m, 0)),
            pl.BlockSpec((8, Cout), lambda b, m: (0, 0)),
            pl.BlockSpec((4, Cout), lambda b, m: (0, 0)),
        ],
        out_specs=pl.BlockSpec((1, MB, Cout), lambda b, m: (b, m, 0)),
        out_shape=jax.ShapeDtypeStruct((B, M, Cout), jnp.float32),
        scratch_shapes=[
            pltpu.VMEM((N, Cout), jnp.float32),
            pltpu.VMEM((8, Cout), jnp.float32),
        ],
        compiler_params=pltpu.CompilerParams(
            dimension_semantics=("arbitrary", "arbitrary")),
    )(zup, zlat, p_t, p_old, stats, gb)

    return (y, p_old)
